# baseline (speedup 1.0000x reference)
import functools
import math

import jax
import jax.numpy as jnp
from jax.experimental import pallas as pl
from jax.experimental.pallas import tpu as pltpu

LANES = 128
G = 16


def _conv1_kernel(x_ref, w_ref, b_ref, w2_ref, b2_ref, fw1_ref, fb1_ref,
                  fw2_ref, fb2_ref, fw3_ref, fb3_ref, o_ref, *, bg):
    accs = []
    for b in range(bg):
        xb = x_ref[b]
        pieces = []
        for j in range(5):
            xr = xb if j == 0 else jnp.concatenate(
                [xb[:, j:], xb[:, :j]], axis=1)
            xtj = jnp.transpose(xr)
            xtj = jnp.concatenate(
                [xtj, jnp.zeros((1024, 80), xtj.dtype)], axis=1)
            for i in range(5):
                pieces.append(xtj[32 * i: 32 * i + 896, :])
        xcat = jnp.concatenate(pieces, axis=1)
        acc = jnp.dot(xcat, w_ref[...], preferred_element_type=jnp.float32)
        accs.append(jnp.maximum(acc + b_ref[...], 0.0).astype(jnp.bfloat16))
    acc = jnp.stack(accs, axis=0)

    a = acc.reshape(bg, 448, 2, LANES)
    pa = jnp.maximum(a[:, :, 0], a[:, :, 1])
    pa = jnp.concatenate([pa, jnp.zeros((bg, 16, LANES), pa.dtype)], axis=1)
    pb = jnp.maximum(pa[:, 0:448], pa[:, 16:464])
    pb = pb.reshape(bg, 14, 2, 16, LANES)[:, :, 0]
    y1 = pb.reshape(bg, 224, LANES)

    t = _conv2_body(y1, w2_ref, b2_ref, bg=bg)

    z = t.reshape(bg * 16, 16 * LANES)
    h = jnp.dot(z, fw1_ref[...], preferred_element_type=jnp.float32)
    h = jnp.maximum(h + fb1_ref[...], 0.0).astype(jnp.bfloat16)
    h = jnp.dot(h, fw2_ref[...], preferred_element_type=jnp.float32)
    h = jnp.maximum(h + fb2_ref[...], 0.0).astype(jnp.bfloat16)
    h = jnp.dot(h, fw3_ref[...], preferred_element_type=jnp.float32)
    o_ref[...] = h + fb3_ref[...]


def _conv2_body(xin, w2_ref, b2_ref, *, bg):
    hw = 224 * bg
    x = jnp.concatenate(
        [xin.reshape(hw, LANES),
         jnp.zeros((72, LANES), xin.dtype)], axis=0)

    acc = jnp.zeros((hw, 2 * LANES), jnp.float32)
    for j in range(5):
        xj = x[j: j + hw + 64, :]
        xcat = jnp.concatenate(
            [xj[16 * i: 16 * i + hw, :] for i in range(5)], axis=1)
        acc = acc + jnp.dot(
            xcat, w2_ref[j], preferred_element_type=jnp.float32)
    acc = jnp.maximum(acc + b2_ref[...], 0.0).astype(jnp.bfloat16)

    a = acc.reshape(bg, 112, 2, 2 * LANES)[:, :80]
    pa = jnp.maximum(a[:, :, 0], a[:, :, 1])
    pa = jnp.concatenate([pa, jnp.zeros((bg, 8, 2 * LANES), pa.dtype)], axis=1)
    pb = jnp.maximum(pa[:, 0:80], pa[:, 8:88])
    pb = pb.reshape(bg, 5, 2, 8, 2 * LANES)[:, :, 0]
    pooled = pb.reshape(bg, 40, 2 * LANES)

    t = jnp.stack([jnp.transpose(pooled[b]) for b in range(bg)], axis=0)
    t = jnp.concatenate(
        [t, jnp.zeros((bg, 2 * LANES, 88), t.dtype)], axis=2)
    return t


def _fc_stack_kernel(x_ref, w1_ref, b1_ref, w2_ref, b2_ref, w3_ref, b3_ref,
                     o_ref):
    h = jnp.dot(x_ref[...], w1_ref[...], preferred_element_type=jnp.float32)
    h = jnp.maximum(h + b1_ref[...], 0.0).astype(jnp.bfloat16)
    h = jnp.dot(h, w2_ref[...], preferred_element_type=jnp.float32)
    h = jnp.maximum(h + b2_ref[...], 0.0).astype(jnp.bfloat16)
    h = jnp.dot(h, w3_ref[...], preferred_element_type=jnp.float32)
    o_ref[...] = h + b3_ref[...]


def _run_net(x, w, b, w2, b2, fw1, fb1, fw2, fb2, fw3, fb3, bg):
    g = x.shape[0]
    body = functools.partial(_conv1_kernel, bg=bg)
    return pl.pallas_call(
        body,
        out_shape=jax.ShapeDtypeStruct((g * G, LANES), jnp.float32),
        grid=(g // bg,),
        in_specs=[
            pl.BlockSpec((bg, 48, 1024), lambda i: (i, 0, 0)),
            pl.BlockSpec((25 * LANES, LANES), lambda i: (0, 0)),
            pl.BlockSpec((1, LANES), lambda i: (0, 0)),
            pl.BlockSpec((5, 5 * LANES, 2 * LANES), lambda i: (0, 0, 0)),
            pl.BlockSpec((1, 2 * LANES), lambda i: (0, 0)),
            pl.BlockSpec(fw1.shape, lambda i: (0, 0)),
            pl.BlockSpec((1, LANES), lambda i: (0, 0)),
            pl.BlockSpec(fw2.shape, lambda i: (0, 0)),
            pl.BlockSpec((1, LANES), lambda i: (0, 0)),
            pl.BlockSpec(fw3.shape, lambda i: (0, 0)),
            pl.BlockSpec((1, LANES), lambda i: (0, 0)),
        ],
        out_specs=pl.BlockSpec((bg * G, LANES), lambda i: (i, 0)),
        compiler_params=pltpu.CompilerParams(
            dimension_semantics=("arbitrary",),
            vmem_limit_bytes=100 * 1024 * 1024,
        ),
    )(x, w, b, w2, b2, fw1, fb1, fw2, fb2, fw3, fb3)


def _run_fc(z, w1, b1, w2, b2, w3, b3, bm):
    m, k = z.shape
    return pl.pallas_call(
        _fc_stack_kernel,
        out_shape=jax.ShapeDtypeStruct((m, LANES), jnp.float32),
        grid=(m // bm,),
        in_specs=[
            pl.BlockSpec((bm, k), lambda i: (i, 0)),
            pl.BlockSpec(w1.shape, lambda i: (0, 0)),
            pl.BlockSpec((1, LANES), lambda i: (0, 0)),
            pl.BlockSpec(w2.shape, lambda i: (0, 0)),
            pl.BlockSpec((1, LANES), lambda i: (0, 0)),
            pl.BlockSpec(w3.shape, lambda i: (0, 0)),
            pl.BlockSpec((1, LANES), lambda i: (0, 0)),
        ],
        out_specs=pl.BlockSpec((bm, LANES), lambda i: (i, 0)),
        compiler_params=pltpu.CompilerParams(
            dimension_semantics=("arbitrary",),
        ),
    )(z, w1, b1, w2, b2, w3, b3)


def _prep_conv_blockdiag(w, b, n_lanes_out, flat=False):
    oc, c, kh, kw = w.shape
    wt = jnp.transpose(w, (2, 3, 1, 0)).reshape(kh * kw, c, oc)
    eye = jnp.eye(G, dtype=w.dtype)
    wb = jnp.einsum("tco,ij->ticjo", wt, eye).reshape(kh * kw, G * c, G * oc)
    wb = jnp.pad(wb, ((0, 0), (0, LANES - G * c),
                      (0, n_lanes_out - G * oc)))
    wb = wb.reshape(kh, kw, LANES, n_lanes_out).transpose(1, 0, 2, 3)
    wb = wb.reshape(kw, kh * LANES, n_lanes_out)
    if flat:
        wb = wb.reshape(kw * kh * LANES, n_lanes_out)
    bp = jnp.pad(jnp.tile(b, G), (0, n_lanes_out - G * oc)).reshape(
        1, n_lanes_out)
    return wb.astype(jnp.bfloat16), bp


def _prep_fc1(w, b):
    wt = w.T.reshape(16, 5, 5, 120)
    wt = jnp.pad(wt, ((0, 0), (0, 0), (0, 3), (0, 0)))
    wt = jnp.pad(wt.reshape(16, 40, 120), ((0, 0), (0, 88), (0, 0)))
    wt = wt.reshape(2048, 120)
    wt = jnp.pad(wt, ((0, 0), (0, LANES - 120)))
    bp = jnp.pad(b, (0, LANES - 120)).reshape(1, LANES)
    return wt.astype(jnp.bfloat16), bp


def _prep_fc(w, b, k_pad):
    out_f, in_f = w.shape
    wt = jnp.pad(w.T, ((0, k_pad - in_f), (0, LANES - out_f)))
    bp = jnp.pad(b, (0, LANES - out_f)).reshape(1, LANES)
    return wt.astype(jnp.bfloat16), bp


def kernel(conv1_w, conv1_b, conv2_w, conv2_b, fc1_w, fc1_b, fc2_w, fc2_b,
           fc3_w, fc3_b, x):
    n = x.shape[0]
    g = n // G
    bg = math.gcd(g, 4)

    xb = x.astype(jnp.bfloat16).reshape(g, G * 3, 1024)

    w1, b1 = _prep_conv_blockdiag(conv1_w, conv1_b, LANES, flat=True)
    w2, b2 = _prep_conv_blockdiag(conv2_w, conv2_b, 2 * LANES)

    fw1, fb1 = _prep_fc1(fc1_w, fc1_b)
    fw2, fb2 = _prep_fc(fc2_w, fc2_b, k_pad=LANES)
    fw3, fb3 = _prep_fc(fc3_w, fc3_b, k_pad=LANES)

    out = _run_net(xb, w1, b1, w2, b2, fw1, fb1, fw2, fb2, fw3, fb3, bg)
    return out[:, :10]


# speedup vs baseline: 1.3081x; 1.3081x over previous
import functools
import math

import jax
import jax.numpy as jnp
from jax.experimental import pallas as pl
from jax.experimental.pallas import tpu as pltpu

LANES = 128
G = 16


def _conv1_kernel(x_ref, w_ref, b_ref, w2_ref, b2_ref, fw1_ref, fb1_ref,
                  fw2_ref, fb2_ref, fw3_ref, fb3_ref, o_ref, *, bg):
    accs = []
    for b in range(bg):
        xb = x_ref[b]
        pieces = []
        for j in range(5):
            xr = xb if j == 0 else jnp.concatenate(
                [xb[:, j:], xb[:, :j]], axis=1)
            xtj = jnp.transpose(xr)
            xtj = jnp.concatenate(
                [xtj, jnp.zeros((1024, 80), xtj.dtype)], axis=1)
            for i in range(5):
                pieces.append(xtj[32 * i: 32 * i + 896, :])
        xcat = jnp.concatenate(pieces, axis=1)
        acc = jnp.dot(xcat, w_ref[...], preferred_element_type=jnp.float32)
        accs.append(jnp.maximum(acc + b_ref[...], 0.0))
    acc = jnp.stack(accs, axis=0)

    a = acc.reshape(bg, 448, 2, LANES)
    pa = jnp.maximum(a[:, :, 0], a[:, :, 1])
    pa = jnp.concatenate([pa, jnp.zeros((bg, 16, LANES), pa.dtype)], axis=1)
    pb = jnp.maximum(pa[:, 0:448], pa[:, 16:464])
    pb = pb.reshape(bg, 14, 2, 16, LANES)[:, :, 0]
    y1 = pb.reshape(bg, 224, LANES).astype(jnp.bfloat16)

    t = _conv2_body(y1, w2_ref, b2_ref, bg=bg)

    z = t.reshape(bg * 16, 16 * LANES)
    h = jnp.dot(z, fw1_ref[...], preferred_element_type=jnp.float32)
    h = jnp.maximum(h + fb1_ref[...], 0.0).astype(jnp.bfloat16)
    h = jnp.dot(h, fw2_ref[...], preferred_element_type=jnp.float32)
    h = jnp.maximum(h + fb2_ref[...], 0.0).astype(jnp.bfloat16)
    h = jnp.dot(h, fw3_ref[...], preferred_element_type=jnp.float32)
    o_ref[...] = h + fb3_ref[...]


def _conv2_body(xin, w2_ref, b2_ref, *, bg):
    hw = 224 * bg
    x = jnp.concatenate(
        [xin.reshape(hw, LANES),
         jnp.zeros((72, LANES), xin.dtype)], axis=0)

    acc = jnp.zeros((hw, 2 * LANES), jnp.float32)
    for j in range(5):
        xj = x[j: j + hw + 64, :]
        xcat = jnp.concatenate(
            [xj[16 * i: 16 * i + hw, :] for i in range(5)], axis=1)
        acc = acc + jnp.dot(
            xcat, w2_ref[j], preferred_element_type=jnp.float32)
    acc = jnp.maximum(acc + b2_ref[...], 0.0)

    a = acc.reshape(bg, 112, 2, 2 * LANES)[:, :80]
    pa = jnp.maximum(a[:, :, 0], a[:, :, 1])
    pa = jnp.concatenate([pa, jnp.zeros((bg, 8, 2 * LANES), pa.dtype)], axis=1)
    pb = jnp.maximum(pa[:, 0:80], pa[:, 8:88])
    pb = pb.reshape(bg, 5, 2, 8, 2 * LANES)[:, :, 0]
    pooled = pb.reshape(bg, 40, 2 * LANES)

    t = jnp.stack([jnp.transpose(pooled[b]) for b in range(bg)], axis=0)
    t = jnp.concatenate(
        [t, jnp.zeros((bg, 2 * LANES, 88), t.dtype)], axis=2)
    return t.astype(jnp.bfloat16)


def _fc_stack_kernel(x_ref, w1_ref, b1_ref, w2_ref, b2_ref, w3_ref, b3_ref,
                     o_ref):
    h = jnp.dot(x_ref[...], w1_ref[...], preferred_element_type=jnp.float32)
    h = jnp.maximum(h + b1_ref[...], 0.0).astype(jnp.bfloat16)
    h = jnp.dot(h, w2_ref[...], preferred_element_type=jnp.float32)
    h = jnp.maximum(h + b2_ref[...], 0.0).astype(jnp.bfloat16)
    h = jnp.dot(h, w3_ref[...], preferred_element_type=jnp.float32)
    o_ref[...] = h + b3_ref[...]


def _run_net(x, w, b, w2, b2, fw1, fb1, fw2, fb2, fw3, fb3, bg):
    g = x.shape[0]
    body = functools.partial(_conv1_kernel, bg=bg)
    return pl.pallas_call(
        body,
        out_shape=jax.ShapeDtypeStruct((g * G, LANES), jnp.float32),
        grid=(g // bg,),
        in_specs=[
            pl.BlockSpec((bg, 48, 1024), lambda i: (i, 0, 0)),
            pl.BlockSpec((25 * LANES, LANES), lambda i: (0, 0)),
            pl.BlockSpec((1, LANES), lambda i: (0, 0)),
            pl.BlockSpec((5, 5 * LANES, 2 * LANES), lambda i: (0, 0, 0)),
            pl.BlockSpec((1, 2 * LANES), lambda i: (0, 0)),
            pl.BlockSpec(fw1.shape, lambda i: (0, 0)),
            pl.BlockSpec((1, LANES), lambda i: (0, 0)),
            pl.BlockSpec(fw2.shape, lambda i: (0, 0)),
            pl.BlockSpec((1, LANES), lambda i: (0, 0)),
            pl.BlockSpec(fw3.shape, lambda i: (0, 0)),
            pl.BlockSpec((1, LANES), lambda i: (0, 0)),
        ],
        out_specs=pl.BlockSpec((bg * G, LANES), lambda i: (i, 0)),
        compiler_params=pltpu.CompilerParams(
            dimension_semantics=("arbitrary",),
            vmem_limit_bytes=100 * 1024 * 1024,
        ),
    )(x, w, b, w2, b2, fw1, fb1, fw2, fb2, fw3, fb3)


def _run_fc(z, w1, b1, w2, b2, w3, b3, bm):
    m, k = z.shape
    return pl.pallas_call(
        _fc_stack_kernel,
        out_shape=jax.ShapeDtypeStruct((m, LANES), jnp.float32),
        grid=(m // bm,),
        in_specs=[
            pl.BlockSpec((bm, k), lambda i: (i, 0)),
            pl.BlockSpec(w1.shape, lambda i: (0, 0)),
            pl.BlockSpec((1, LANES), lambda i: (0, 0)),
            pl.BlockSpec(w2.shape, lambda i: (0, 0)),
            pl.BlockSpec((1, LANES), lambda i: (0, 0)),
            pl.BlockSpec(w3.shape, lambda i: (0, 0)),
            pl.BlockSpec((1, LANES), lambda i: (0, 0)),
        ],
        out_specs=pl.BlockSpec((bm, LANES), lambda i: (i, 0)),
        compiler_params=pltpu.CompilerParams(
            dimension_semantics=("arbitrary",),
        ),
    )(z, w1, b1, w2, b2, w3, b3)


def _prep_conv_blockdiag(w, b, n_lanes_out, flat=False):
    oc, c, kh, kw = w.shape
    wt = jnp.transpose(w, (2, 3, 1, 0)).reshape(kh * kw, c, oc)
    eye = jnp.eye(G, dtype=w.dtype)
    wb = jnp.einsum("tco,ij->ticjo", wt, eye).reshape(kh * kw, G * c, G * oc)
    wb = jnp.pad(wb, ((0, 0), (0, LANES - G * c),
                      (0, n_lanes_out - G * oc)))
    wb = wb.reshape(kh, kw, LANES, n_lanes_out).transpose(1, 0, 2, 3)
    wb = wb.reshape(kw, kh * LANES, n_lanes_out)
    if flat:
        wb = wb.reshape(kw * kh * LANES, n_lanes_out)
    bp = jnp.pad(jnp.tile(b, G), (0, n_lanes_out - G * oc)).reshape(
        1, n_lanes_out)
    return wb.astype(jnp.bfloat16), bp


def _prep_fc1(w, b):
    wt = w.T.reshape(16, 5, 5, 120)
    wt = jnp.pad(wt, ((0, 0), (0, 0), (0, 3), (0, 0)))
    wt = jnp.pad(wt.reshape(16, 40, 120), ((0, 0), (0, 88), (0, 0)))
    wt = wt.reshape(2048, 120)
    wt = jnp.pad(wt, ((0, 0), (0, LANES - 120)))
    bp = jnp.pad(b, (0, LANES - 120)).reshape(1, LANES)
    return wt.astype(jnp.bfloat16), bp


def _prep_fc(w, b, k_pad):
    out_f, in_f = w.shape
    wt = jnp.pad(w.T, ((0, k_pad - in_f), (0, LANES - out_f)))
    bp = jnp.pad(b, (0, LANES - out_f)).reshape(1, LANES)
    return wt.astype(jnp.bfloat16), bp


def kernel(conv1_w, conv1_b, conv2_w, conv2_b, fc1_w, fc1_b, fc2_w, fc2_b,
           fc3_w, fc3_b, x):
    n = x.shape[0]
    g = n // G
    bg = math.gcd(g, 8)

    xb = x.astype(jnp.bfloat16).reshape(g, G * 3, 1024)

    w1, b1 = _prep_conv_blockdiag(conv1_w, conv1_b, LANES, flat=True)
    w2, b2 = _prep_conv_blockdiag(conv2_w, conv2_b, 2 * LANES)

    fw1, fb1 = _prep_fc1(fc1_w, fc1_b)
    fw2, fb2 = _prep_fc(fc2_w, fc2_b, k_pad=LANES)
    fw3, fb3 = _prep_fc(fc3_w, fc3_b, k_pad=LANES)

    out = _run_net(xb, w1, b1, w2, b2, fw1, fb1, fw2, fb2, fw3, fb3, bg)
    return out[:, :10]
